# baseline (speedup 1.0000x reference)
import functools

import jax
import jax.numpy as jnp
from jax import lax
from jax.experimental import pallas as pl
from jax.experimental.pallas import tpu as pltpu
from jax.experimental.pallas import tpu_sc as plsc

NUM_TABLES = 26
VOCAB_SZ = 100000
BATCH_SZ = 16384

NUM_CORES = 2
NUM_SUBCORES = 16
LANES = 16

MAXC = 16
ROUNDS = 4
RND_ROWS = BATCH_SZ // ROUNDS
ROWS_W = RND_ROWS // NUM_SUBCORES
P1_ROWS = 128


def _sc_half(col0, ncols, sid,
             tables_hbm, idx_hbm, out_hbm,
             table_v, ibuf, tbuf, fsrc, fdst, p2i, p2f,
             shared_idx, shared_out, sem_t):
    lane = lax.iota(jnp.int32, LANES)

    @pl.when(sid < ncols)
    def _():
        pltpu.async_copy(tables_hbm.at[col0 + sid], table_v, sem_t)

    for h in range(ROUNDS):
        rows0 = h * RND_ROWS + sid * ROWS_W

        for r in range(ROWS_W // P1_ROWS):
            pltpu.sync_copy(
                idx_hbm.at[pl.ds(rows0 + r * P1_ROWS, P1_ROWS), :], ibuf)

            @pl.loop(0, ncols * (P1_ROWS // LANES), unroll=8)
            def _p1(i):
                k = lax.div(i, P1_ROWS // LANES)
                v = lax.rem(i, P1_ROWS // LANES)
                rows = v * LANES + lane
                cols = jnp.full((LANES,), col0 + k, jnp.int32)
                tbuf[k, pl.ds(v * LANES, LANES)] = plsc.load_gather(
                    ibuf, [rows, cols])

            pltpu.sync_copy(
                tbuf,
                shared_idx.at[:, pl.ds(sid * ROWS_W + r * P1_ROWS, P1_ROWS)])

        plsc.subcore_barrier()

        @pl.when(sid < ncols)
        def _():
            if h == 0:
                pltpu.make_async_copy(
                    tables_hbm.at[col0 + sid], table_v, sem_t).wait()
            pltpu.sync_copy(shared_idx.at[sid], p2i)

            @pl.loop(0, RND_ROWS // LANES, unroll=8)
            def _p2(i):
                sl = pl.ds(i * LANES, LANES)
                p2f[sl] = plsc.load_gather(table_v, [p2i[sl]])

            pltpu.sync_copy(p2f, shared_out.at[sid])

        plsc.subcore_barrier()

        pltpu.sync_copy(shared_out.at[:, pl.ds(sid * ROWS_W, ROWS_W)], fsrc)

        @pl.loop(0, ncols * (ROWS_W // LANES), unroll=8)
        def _p3(i):
            k = lax.div(i, ROWS_W // LANES)
            m = lax.rem(i, ROWS_W // LANES)
            g = fsrc[k, pl.ds(m * LANES, LANES)]
            rows = m * LANES + lane
            cols = jnp.full((LANES,), k, jnp.int32)
            plsc.store_scatter(fdst, [rows, cols], g)

        pltpu.sync_copy(
            fdst, out_hbm.at[pl.ds(rows0, ROWS_W), pl.ds(col0, MAXC)])


def _emb_body(tables_hbm, idx_hbm, out_hbm, *scratch):
    cid = lax.axis_index("c")
    sid = lax.axis_index("s")

    @pl.when(cid == 0)
    def _():
        _sc_half(0, MAXC, sid, tables_hbm, idx_hbm, out_hbm, *scratch)

    @pl.when(cid != 0)
    def _():
        _sc_half(MAXC, NUM_TABLES - MAXC, sid,
                 tables_hbm, idx_hbm, out_hbm, *scratch)


@functools.partial(
    pl.kernel,
    out_type=jax.ShapeDtypeStruct((BATCH_SZ, 2 * MAXC), jnp.float32),
    mesh=plsc.VectorSubcoreMesh(core_axis_name="c", subcore_axis_name="s"),
    scratch_types=[
        pltpu.VMEM((VOCAB_SZ,), jnp.float32),
        pltpu.VMEM((P1_ROWS, NUM_TABLES), jnp.int32),
        pltpu.VMEM((MAXC, P1_ROWS), jnp.int32),
        pltpu.VMEM((MAXC, ROWS_W), jnp.float32),
        pltpu.VMEM((ROWS_W, MAXC), jnp.float32),
        pltpu.VMEM((RND_ROWS,), jnp.int32),
        pltpu.VMEM((RND_ROWS,), jnp.float32),
        pltpu.VMEM_SHARED((MAXC, RND_ROWS), jnp.int32),
        pltpu.VMEM_SHARED((MAXC, RND_ROWS), jnp.float32),
        pltpu.SemaphoreType.DMA,
    ],
    compiler_params=pltpu.CompilerParams(
        needs_layout_passes=False, use_tc_tiling_on_sc=False),
)
def _emb_kernel(tables_hbm, idx_hbm, out_hbm, *scratch):
    _emb_body(tables_hbm, idx_hbm, out_hbm, *scratch)


def kernel(categorical_tensor, tables):
    idx = categorical_tensor.astype(jnp.int32)
    tables2 = tables.reshape(NUM_TABLES, VOCAB_SZ)
    out_pad = _emb_kernel(tables2, idx)
    return out_pad[:, :NUM_TABLES]


# speedup vs baseline: 1.0021x; 1.0021x over previous
import functools

import jax
import jax.numpy as jnp
from jax import lax
from jax.experimental import pallas as pl
from jax.experimental.pallas import tpu as pltpu
from jax.experimental.pallas import tpu_sc as plsc

NUM_TABLES = 26
VOCAB_SZ = 100000
BATCH_SZ = 16384

NUM_CORES = 2
NUM_SUBCORES = 16
LANES = 16

MAXC = 16
ROUNDS = 4
RND_ROWS = BATCH_SZ // ROUNDS
ROWS_W = RND_ROWS // NUM_SUBCORES
P1_ROWS = 128


def _sc_half(col0, ncols, sid,
             tables_hbm, idx_hbm, out_hbm,
             table_v, ibuf, tbuf, fsrc, fdst, p2i, p2f,
             shared_idx, shared_out, sem_t):
    lane = lax.iota(jnp.int32, LANES)

    @pl.when(sid < ncols)
    def _():
        pltpu.async_copy(tables_hbm.at[col0 + sid], table_v, sem_t)

    for h in range(ROUNDS):
        rows0 = h * RND_ROWS + sid * ROWS_W

        for r in range(ROWS_W // P1_ROWS):
            pltpu.sync_copy(
                idx_hbm.at[pl.ds(rows0 + r * P1_ROWS, P1_ROWS), :], ibuf)

            @pl.loop(0, ncols * (P1_ROWS // LANES), unroll=8)
            def _p1(i):
                k = lax.div(i, P1_ROWS // LANES)
                v = lax.rem(i, P1_ROWS // LANES)
                rows = v * LANES + lane
                cols = jnp.full((LANES,), col0 + k, jnp.int32)
                tbuf[k, pl.ds(v * LANES, LANES)] = plsc.load_gather(
                    ibuf, [rows, cols])

            pltpu.sync_copy(
                tbuf,
                shared_idx.at[:, pl.ds(sid * ROWS_W + r * P1_ROWS, P1_ROWS)])

        plsc.subcore_barrier()

        @pl.when(sid < ncols)
        def _():
            if h == 0:
                pltpu.make_async_copy(
                    tables_hbm.at[col0 + sid], table_v, sem_t).wait()
            pltpu.sync_copy(shared_idx.at[sid], p2i)

            @pl.loop(0, RND_ROWS // LANES, unroll=8)
            def _p2(i):
                sl = pl.ds(i * LANES, LANES)
                p2f[sl] = plsc.load_gather(table_v, [p2i[sl]])

            pltpu.sync_copy(p2f, shared_out.at[sid])

        plsc.subcore_barrier()

        pltpu.sync_copy(shared_out.at[:, pl.ds(sid * ROWS_W, ROWS_W)], fsrc)

        @pl.loop(0, ncols * (ROWS_W // LANES), unroll=8)
        def _p3(i):
            k = lax.div(i, ROWS_W // LANES)
            m = lax.rem(i, ROWS_W // LANES)
            g = fsrc[k, pl.ds(m * LANES, LANES)]
            rows = m * LANES + lane
            cols = jnp.full((LANES,), k, jnp.int32)
            plsc.store_scatter(fdst, [rows, cols], g)

        pltpu.sync_copy(
            fdst, out_hbm.at[pl.ds(rows0, ROWS_W), pl.ds(col0, MAXC)])


def _emb_body(tables_hbm, idx_hbm, out_hbm, *scratch):
    cid = lax.axis_index("c")
    sid = lax.axis_index("s")

    @pl.when(cid == 0)
    def _():
        _sc_half(0, MAXC, sid, tables_hbm, idx_hbm, out_hbm, *scratch)

    @pl.when(cid != 0)
    def _():
        _sc_half(MAXC, NUM_TABLES - MAXC, sid,
                 tables_hbm, idx_hbm, out_hbm, *scratch)


@functools.partial(
    pl.kernel,
    out_type=jax.ShapeDtypeStruct((BATCH_SZ, 2 * MAXC), jnp.float32),
    mesh=plsc.VectorSubcoreMesh(core_axis_name="c", subcore_axis_name="s"),
    scratch_types=[
        pltpu.VMEM((VOCAB_SZ,), jnp.float32),
        pltpu.VMEM((P1_ROWS, NUM_TABLES), jnp.int32),
        pltpu.VMEM((MAXC, P1_ROWS), jnp.int32),
        pltpu.VMEM((MAXC, ROWS_W), jnp.float32),
        pltpu.VMEM((ROWS_W, MAXC), jnp.float32),
        pltpu.VMEM((RND_ROWS,), jnp.int32),
        pltpu.VMEM((RND_ROWS,), jnp.float32),
        pltpu.VMEM_SHARED((MAXC, RND_ROWS), jnp.int32),
        pltpu.VMEM_SHARED((MAXC, RND_ROWS), jnp.float32),
        pltpu.SemaphoreType.DMA,
    ],
    compiler_params=pltpu.CompilerParams(
        needs_layout_passes=False, use_tc_tiling_on_sc=False),
)
def _emb_kernel(tables_hbm, idx_hbm, out_hbm, *scratch):
    _emb_body(tables_hbm, idx_hbm, out_hbm, *scratch)


def kernel(categorical_tensor, tables):
    idx = categorical_tensor.astype(jnp.int32)
    tables2 = tables.reshape(NUM_TABLES, VOCAB_SZ)
    out_pad = _emb_kernel(tables2, idx)
    return out_pad


# speedup vs baseline: 1.3806x; 1.3777x over previous
import functools

import jax
import jax.numpy as jnp
from jax import lax
from jax.experimental import pallas as pl
from jax.experimental.pallas import tpu as pltpu
from jax.experimental.pallas import tpu_sc as plsc

NUM_TABLES = 26
VOCAB_SZ = 100000
BATCH_SZ = 16384

NUM_CORES = 2
NUM_SUBCORES = 16
LANES = 16

CHUNK = 4096
NCHUNK = BATCH_SZ // CHUNK


def _emb_body(tables_hbm, idx_hbm, out_hbm,
              table_v, ibuf0, ibuf1, obuf0, obuf1, sem_t, sem_i, sem_o):
    wid = lax.axis_index("s") * NUM_CORES + lax.axis_index("c")

    @pl.when(wid < NUM_TABLES)
    def _():
        ibufs = (ibuf0, ibuf1)
        obufs = (obuf0, obuf1)
        tdesc = pltpu.async_copy(
            tables_hbm.at[pl.ds(wid * VOCAB_SZ, VOCAB_SZ)], table_v, sem_t)
        descs_i = [
            pltpu.async_copy(
                idx_hbm.at[wid, pl.ds(q * CHUNK, CHUNK)], ibufs[q], sem_i)
            for q in range(2)
        ]
        tdesc.wait()
        descs_o = []
        for q in range(NCHUNK):
            ib, ob = ibufs[q % 2], obufs[q % 2]
            descs_i[q].wait()
            if q >= 2:
                descs_o[q - 2].wait()

            @pl.loop(0, CHUNK // LANES, unroll=8)
            def _gather(i):
                sl = pl.ds(i * LANES, LANES)
                ob[sl] = plsc.load_gather(table_v, [ib[sl]])

            descs_o.append(pltpu.async_copy(
                ob, out_hbm.at[wid, pl.ds(q * CHUNK, CHUNK)], sem_o))
            if q + 2 < NCHUNK:
                descs_i.append(pltpu.async_copy(
                    idx_hbm.at[wid, pl.ds((q + 2) * CHUNK, CHUNK)],
                    ib, sem_i))
        descs_o[NCHUNK - 2].wait()
        descs_o[NCHUNK - 1].wait()


@functools.partial(
    pl.kernel,
    out_type=jax.ShapeDtypeStruct((NUM_TABLES, BATCH_SZ), jnp.float32),
    mesh=plsc.VectorSubcoreMesh(core_axis_name="c", subcore_axis_name="s"),
    scratch_types=[
        pltpu.VMEM((VOCAB_SZ,), jnp.float32),
        pltpu.VMEM((CHUNK,), jnp.int32),
        pltpu.VMEM((CHUNK,), jnp.int32),
        pltpu.VMEM((CHUNK,), jnp.float32),
        pltpu.VMEM((CHUNK,), jnp.float32),
        pltpu.SemaphoreType.DMA,
        pltpu.SemaphoreType.DMA,
        pltpu.SemaphoreType.DMA,
    ],
    compiler_params=pltpu.CompilerParams(needs_layout_passes=False),
)
def _emb_kernel(tables_hbm, idx_hbm, out_hbm, *scratch):
    _emb_body(tables_hbm, idx_hbm, out_hbm, *scratch)


def kernel(categorical_tensor, tables):
    idx_t = categorical_tensor.astype(jnp.int32).T
    out_t = _emb_kernel(tables.reshape(NUM_TABLES * VOCAB_SZ), idx_t)
    return out_t.T


# speedup vs baseline: 4.4277x; 3.2071x over previous
import functools

import jax
import jax.numpy as jnp
from jax import lax
from jax.experimental import pallas as pl
from jax.experimental.pallas import tpu as pltpu
from jax.experimental.pallas import tpu_sc as plsc

NUM_TABLES = 26
VOCAB_SZ = 100000
BATCH_SZ = 16384

NUM_CORES = 2
NUM_SUBCORES = 16
LANES = 16

CHUNK = 4096
NCHUNK = BATCH_SZ // CHUNK


def _emb_body(tables_hbm, idx_hbm, out_hbm,
              table_v, ibuf0, ibuf1, obuf0, obuf1, sem_t, sem_i, sem_o):
    wid = lax.axis_index("s") * NUM_CORES + lax.axis_index("c")

    @pl.when(wid < NUM_TABLES)
    def _():
        ibufs = (ibuf0, ibuf1)
        obufs = (obuf0, obuf1)
        tdesc = pltpu.async_copy(tables_hbm.at[wid], table_v, sem_t)
        descs_i = [
            pltpu.async_copy(
                idx_hbm.at[wid, pl.ds(q * CHUNK, CHUNK)], ibufs[q], sem_i)
            for q in range(2)
        ]
        tdesc.wait()
        descs_o = []
        for q in range(NCHUNK):
            ib, ob = ibufs[q % 2], obufs[q % 2]
            descs_i[q].wait()
            if q >= 2:
                descs_o[q - 2].wait()

            @pl.loop(0, CHUNK // LANES, unroll=8)
            def _gather(i):
                sl = pl.ds(i * LANES, LANES)
                ob[sl] = plsc.load_gather(table_v, [ib[sl]])

            descs_o.append(pltpu.async_copy(
                ob, out_hbm.at[wid, pl.ds(q * CHUNK, CHUNK)], sem_o))
            if q + 2 < NCHUNK:
                descs_i.append(pltpu.async_copy(
                    idx_hbm.at[wid, pl.ds((q + 2) * CHUNK, CHUNK)],
                    ib, sem_i))
        descs_o[NCHUNK - 2].wait()
        descs_o[NCHUNK - 1].wait()


@functools.partial(
    pl.kernel,
    out_type=jax.ShapeDtypeStruct((NUM_TABLES, BATCH_SZ), jnp.float32),
    mesh=plsc.VectorSubcoreMesh(core_axis_name="c", subcore_axis_name="s"),
    scratch_types=[
        pltpu.VMEM((VOCAB_SZ,), jnp.float32),
        pltpu.VMEM((CHUNK,), jnp.int32),
        pltpu.VMEM((CHUNK,), jnp.int32),
        pltpu.VMEM((CHUNK,), jnp.float32),
        pltpu.VMEM((CHUNK,), jnp.float32),
        pltpu.SemaphoreType.DMA,
        pltpu.SemaphoreType.DMA,
        pltpu.SemaphoreType.DMA,
    ],
    compiler_params=pltpu.CompilerParams(needs_layout_passes=False),
)
def _emb_kernel(tables_hbm, idx_hbm, out_hbm, *scratch):
    _emb_body(tables_hbm, idx_hbm, out_hbm, *scratch)


def kernel(categorical_tensor, tables):
    idx_t = categorical_tensor.astype(jnp.int32).T
    out_t = _emb_kernel(tables.reshape(NUM_TABLES, VOCAB_SZ), idx_t)
    return out_t.T
